# speedup vs baseline: 1.0438x; 1.0438x over previous
import functools

import jax
import jax.numpy as jnp
from jax import lax
from jax.experimental import pallas as pl
from jax.experimental.pallas import tpu as pltpu
from jax.experimental.pallas import tpu_sc as plsc

_NC = 2
_NS = 16
_L = 16
_NW = _NC * _NS


@functools.lru_cache(maxsize=None)
def _make_kernel(B, S, V, D):
    TOT = B * S
    PW = TOT // _NW
    CH = 64
    NCH = PW // CH
    W_PER_SEQ = S // PW

    mesh = plsc.VectorSubcoreMesh(
        core_axis_name="c", subcore_axis_name="s",
        num_cores=_NC, num_subcores=_NS,
    )

    @functools.partial(
        pl.kernel,
        out_type=jax.ShapeDtypeStruct((TOT, D), jnp.float32),
        mesh=mesh,
        scratch_types=[
            pltpu.VMEM((PW,), jnp.int32),
            pltpu.VMEM((CH, D), jnp.float32),
            pltpu.VMEM((CH, D), jnp.float32),
            pltpu.SemaphoreType.DMA,
        ],
    )
    def tok_pos_embed(ids_hbm, tok_hbm, pos_hbm, out_hbm,
                      ids_v, tok_buf, pos_buf, sem):
        wid = lax.axis_index("s") * _NC + lax.axis_index("c")
        base = wid * PW
        pos_base = lax.rem(wid, W_PER_SEQ) * PW
        pltpu.sync_copy(ids_hbm.at[pl.ds(base, PW)], ids_v)
        for c in range(NCH):
            gather = pltpu.async_copy(
                tok_hbm.at[ids_v.at[pl.ds(c * CH, CH)]], tok_buf, sem)
            pltpu.sync_copy(
                pos_hbm.at[pl.ds(pos_base + c * CH, CH)], pos_buf)
            gather.wait()

            def add_row(r, carry):
                for j in range(D // _L):
                    sl = pl.ds(j * _L, _L)
                    tok_buf[r, sl] = tok_buf[r, sl] + pos_buf[r, sl]
                return carry

            lax.fori_loop(0, CH, add_row, 0)
            pltpu.sync_copy(tok_buf, out_hbm.at[pl.ds(base + c * CH, CH)])

    return tok_pos_embed


def kernel(input_ids, token_table, pos_table):
    B, S = input_ids.shape
    V, D = token_table.shape
    ids = input_ids.reshape(B * S).astype(jnp.int32)
    out = _make_kernel(B, S, V, D)(ids, token_table, pos_table)
    return out.reshape(B, S, D)


# speedup vs baseline: 1.0864x; 1.0407x over previous
import functools

import jax
import jax.numpy as jnp
from jax import lax
from jax.experimental import pallas as pl
from jax.experimental.pallas import tpu as pltpu
from jax.experimental.pallas import tpu_sc as plsc

_NC = 2
_NS = 16
_L = 16
_NW = _NC * _NS


@functools.lru_cache(maxsize=None)
def _make_kernel(B, S, V, D):
    POS_W = S // _NW
    CH = 32
    NCH = (B * POS_W) // CH
    H = POS_W // CH

    mesh = plsc.VectorSubcoreMesh(
        core_axis_name="c", subcore_axis_name="s",
        num_cores=_NC, num_subcores=_NS,
    )

    @functools.partial(
        pl.kernel,
        out_type=jax.ShapeDtypeStruct((B * S, D), jnp.float32),
        mesh=mesh,
        scratch_types=[
            pltpu.VMEM((B, POS_W), jnp.int32),
            pltpu.VMEM((POS_W, D), jnp.float32),
            pltpu.VMEM((CH, D), jnp.float32),
            pltpu.VMEM((CH, D), jnp.float32),
            pltpu.SemaphoreType.DMA,
            pltpu.SemaphoreType.DMA,
            pltpu.SemaphoreType.DMA,
            pltpu.SemaphoreType.DMA,
        ],
    )
    def tok_pos_embed(ids_hbm, tok_hbm, pos_hbm, out_hbm,
                      ids_v, pos_buf, tok0, tok1, gsem, psem, wsem, isem):
        wid = lax.axis_index("s") * _NC + lax.axis_index("c")
        p0 = wid * POS_W
        id_cps = [
            pltpu.async_copy(ids_hbm.at[b, pl.ds(p0, POS_W)], ids_v.at[b], isem)
            for b in range(B)
        ]
        pos_cp = pltpu.async_copy(pos_hbm.at[pl.ds(p0, POS_W)], pos_buf, psem)
        for cp in id_cps:
            cp.wait()

        bufs = (tok0, tok1)

        def chunk_idx(k):
            return ids_v.at[k // H, pl.ds((k % H) * CH, CH)]

        gathers = [None] * NCH
        writebacks = [None] * NCH
        gathers[0] = pltpu.async_copy(tok_hbm.at[chunk_idx(0)], bufs[0], gsem)
        pos_cp.wait()

        for k in range(NCH):
            buf = bufs[k % 2]
            if k + 1 < NCH:
                if k >= 1:
                    writebacks[k - 1].wait()
                gathers[k + 1] = pltpu.async_copy(
                    tok_hbm.at[chunk_idx(k + 1)], bufs[(k + 1) % 2], gsem)
            gathers[k].wait()

            off = (k % H) * CH

            def add_row(r, carry, buf=buf, off=off):
                for j in range(D // _L):
                    sl = pl.ds(j * _L, _L)
                    plsc.addupdate(buf.at[r, sl], pos_buf[off + r, sl])
                return carry

            lax.fori_loop(0, CH, add_row, 0)

            out_off = (k // H) * S + p0 + (k % H) * CH
            writebacks[k] = pltpu.async_copy(
                buf, out_hbm.at[pl.ds(out_off, CH)], wsem)

        writebacks[NCH - 2].wait()
        writebacks[NCH - 1].wait()

    return tok_pos_embed


def kernel(input_ids, token_table, pos_table):
    B, S = input_ids.shape
    V, D = token_table.shape
    ids = input_ids.astype(jnp.int32)
    out = _make_kernel(B, S, V, D)(ids, token_table, pos_table)
    return out.reshape(B, S, D)


# speedup vs baseline: 1.7447x; 1.6060x over previous
import functools

import jax
import jax.numpy as jnp
from jax import lax
from jax.experimental import pallas as pl
from jax.experimental.pallas import tpu as pltpu
from jax.experimental.pallas import tpu_sc as plsc

_NC = 2
_NS = 16
_L = 16
_NW = _NC * _NS


@functools.lru_cache(maxsize=None)
def _make_kernel(B, S, V, D):
    POS_W = S // _NW
    CH = 32
    NB = 3
    NCH = (B * POS_W) // CH
    H = POS_W // CH

    mesh = plsc.VectorSubcoreMesh(
        core_axis_name="c", subcore_axis_name="s",
        num_cores=_NC, num_subcores=_NS,
    )

    @functools.partial(
        pl.kernel,
        out_type=jax.ShapeDtypeStruct((B * S, D), jnp.float32),
        mesh=mesh,
        scratch_types=[
            pltpu.VMEM((B, POS_W), jnp.int32),
            pltpu.VMEM((POS_W, D), jnp.float32),
            [pltpu.VMEM((CH, D), jnp.float32)] * NB,
            pltpu.SemaphoreType.DMA,
            pltpu.SemaphoreType.DMA,
            pltpu.SemaphoreType.DMA,
            pltpu.SemaphoreType.DMA,
        ],
    )
    def tok_pos_embed(ids_hbm, tok_hbm, pos_hbm, out_hbm,
                      ids_v, pos_buf, bufs, gsem, psem, wsem, isem):
        wid = lax.axis_index("s") * _NC + lax.axis_index("c")
        p0 = wid * POS_W
        id_cps = [
            pltpu.async_copy(ids_hbm.at[b, pl.ds(p0, POS_W)], ids_v.at[b], isem)
            for b in range(B)
        ]
        pos_cp = pltpu.async_copy(pos_hbm.at[pl.ds(p0, POS_W)], pos_buf, psem)
        for cp in id_cps:
            cp.wait()

        def chunk_idx(k):
            return ids_v.at[k // H, pl.ds((k % H) * CH, CH)]

        def start_gather(k):
            return pltpu.async_copy(tok_hbm.at[chunk_idx(k)], bufs[k % NB], gsem)

        gathers = [None] * NCH
        writebacks = [None] * NCH
        for k in range(NB - 1):
            gathers[k] = start_gather(k)
        pos_cp.wait()

        for k in range(NCH):
            buf = bufs[k % NB]
            gathers[k].wait()

            off = (k % H) * CH

            @functools.partial(plsc.parallel_loop, 0, CH)
            def add_row(r, buf=buf, off=off):
                for j in range(D // _L):
                    sl = pl.ds(j * _L, _L)
                    plsc.addupdate(buf.at[r, sl], pos_buf[off + r, sl])

            out_off = (k // H) * S + p0 + (k % H) * CH
            writebacks[k] = pltpu.async_copy(
                buf, out_hbm.at[pl.ds(out_off, CH)], wsem)

            nxt = k + NB - 1
            if nxt < NCH:
                if nxt - NB >= 0:
                    writebacks[nxt - NB].wait()
                gathers[nxt] = start_gather(nxt)

        for k in range(NCH - NB, NCH):
            if k >= 0:
                writebacks[k].wait()

    return tok_pos_embed


def kernel(input_ids, token_table, pos_table):
    B, S = input_ids.shape
    V, D = token_table.shape
    ids = input_ids.astype(jnp.int32)
    out = _make_kernel(B, S, V, D)(ids, token_table, pos_table)
    return out.reshape(B, S, D)
